# baseline (speedup 1.0000x reference)
import functools

import jax
import jax.numpy as jnp
from jax import lax
from jax.experimental import pallas as pl
from jax.experimental.pallas import tpu as pltpu
from jax.experimental.pallas import tpu_sc as plsc

VSZ = 1000000
DSZ = 64
B_TOTAL = 16384 * 20

NC = 2
NS = 16
NW = NC * NS
B_PER_W = B_TOTAL // NW
CHUNK = 128
NCHUNK = B_PER_W // CHUNK


def _emb_body(idx_hbm, w_hbm, out_hbm, idx_v, rows_v, sem):
    wid = lax.axis_index("s") * NC + lax.axis_index("c")
    base = wid * B_PER_W

    pltpu.sync_copy(idx_hbm.at[wid], idx_v)

    def chunk_body(j, carry):
        pltpu.async_copy(w_hbm.at[idx_v.at[j]], rows_v, sem).wait()

        m_any = idx_v[j, pl.ds(0, 16)] == 0
        for v in range(1, CHUNK // 16):
            m_any = m_any | (idx_v[j, pl.ds(v * 16, 16)] == 0)
        mi = jnp.where(m_any, jnp.ones((16,), jnp.int32),
                       jnp.zeros((16,), jnp.int32))
        npad = mi[0]
        for l in range(1, 16):
            npad = npad | mi[l]

        @pl.when(npad > 0)
        def _fixup():
            zeros16 = jnp.zeros((16,), jnp.float32)
            for v in range(CHUNK // 16):
                iv = idx_v[j, pl.ds(v * 16, 16)]
                for l in range(16):
                    @pl.when(iv[l] == 0)
                    def _zero_row(v=v, l=l):
                        for c in range(DSZ // 16):
                            rows_v[v * 16 + l, pl.ds(c * 16, 16)] = zeros16

        pltpu.sync_copy(rows_v, out_hbm.at[pl.ds(base + j * CHUNK, CHUNK)])
        return carry

    lax.fori_loop(0, NCHUNK, chunk_body, 0)


@jax.jit
def _emb_lookup(idx, w):
    mesh = plsc.VectorSubcoreMesh(core_axis_name="c", subcore_axis_name="s")
    return pl.kernel(
        _emb_body,
        out_type=jax.ShapeDtypeStruct((B_TOTAL, DSZ), jnp.float32),
        mesh=mesh,
        compiler_params=pltpu.CompilerParams(use_tc_tiling_on_sc=False),
        scratch_types=[
            pltpu.VMEM((NCHUNK, CHUNK), jnp.int32),
            pltpu.VMEM((CHUNK, DSZ), jnp.float32),
            pltpu.SemaphoreType.DMA,
        ],
    )(idx, w)


def kernel(x, W):
    idx = x.reshape(NW, NCHUNK, CHUNK).astype(jnp.int32)
    out = _emb_lookup(idx, W)
    return out.reshape(x.shape[0], x.shape[1], DSZ)


# speedup vs baseline: 1.0672x; 1.0672x over previous
import jax
import jax.numpy as jnp
from jax import lax
from jax.experimental import pallas as pl
from jax.experimental.pallas import tpu as pltpu
from jax.experimental.pallas import tpu_sc as plsc

VSZ = 1000000
DSZ = 64
B_TOTAL = 16384 * 20

NC = 2
NS = 16
NW = NC * NS
B_PER_W = B_TOTAL // NW
CHUNK = 128
NCHUNK = B_PER_W // CHUNK
NBUF = 4
G = 3


def _emb_body(idx_hbm, w_hbm, out_hbm, idx_v, rows_v, gsem, ssem):
    wid = lax.axis_index("s") * NC + lax.axis_index("c")
    base = wid * B_PER_W

    pltpu.sync_copy(idx_hbm.at[wid], idx_v)

    def gather(k, b):
        pltpu.async_copy(w_hbm.at[idx_v.at[k]], rows_v.at[b], gsem.at[b])

    def wait_gather(b):
        pltpu.make_async_copy(
            w_hbm.at[idx_v.at[0]], rows_v.at[b], gsem.at[b]).wait()

    def store(j, b):
        pltpu.async_copy(
            rows_v.at[b], out_hbm.at[pl.ds(base + j * CHUNK, CHUNK)],
            ssem.at[b])

    def wait_store(b):
        pltpu.make_async_copy(
            rows_v.at[b], out_hbm.at[pl.ds(base, CHUNK)], ssem.at[b]).wait()

    for k in range(G):
        gather(k, k % NBUF)

    def fixup(j, b):
        m_any = idx_v[j, pl.ds(0, 16)] == 0
        for v in range(1, CHUNK // 16):
            m_any = m_any | (idx_v[j, pl.ds(v * 16, 16)] == 0)
        mi = jnp.where(m_any, jnp.ones((16,), jnp.int32),
                       jnp.zeros((16,), jnp.int32))
        npad = mi[0]
        for l in range(1, 16):
            npad = npad | mi[l]

        @pl.when(npad > 0)
        def _fix():
            zeros16 = jnp.zeros((16,), jnp.float32)

            def per_vreg(v, carry):
                iv = idx_v[j, pl.ds(v * 16, 16)]
                for l in range(16):
                    @pl.when(iv[l] == 0)
                    def _zero_row(v=v, l=l):
                        for c in range(DSZ // 16):
                            rows_v[b, v * 16 + l, pl.ds(c * 16, 16)] = zeros16
                return carry

            lax.fori_loop(0, CHUNK // 16, per_vreg, 0)

    def stage_body(s, carry):
        jb = s * NBUF
        for b in range(NBUF):
            j = jb + b
            wait_gather(b)
            fixup(j, b)
            store(j, b)
            bg = (b + G) % NBUF
            k = j + G

            @pl.when(k < NCHUNK)
            def _prefetch(k=k, bg=bg):
                @pl.when(k >= NBUF)
                def _drain(bg=bg):
                    wait_store(bg)
                gather(k, bg)
        return carry

    lax.fori_loop(0, NCHUNK // NBUF, stage_body, 0)

    for i in range(G):
        wait_store((NCHUNK - G + i) % NBUF)


@jax.jit
def _emb_lookup(idx, w):
    mesh = plsc.VectorSubcoreMesh(core_axis_name="c", subcore_axis_name="s")
    return pl.kernel(
        _emb_body,
        out_type=jax.ShapeDtypeStruct((B_TOTAL, DSZ), jnp.float32),
        mesh=mesh,
        compiler_params=pltpu.CompilerParams(use_tc_tiling_on_sc=False),
        scratch_types=[
            pltpu.VMEM((NCHUNK, CHUNK), jnp.int32),
            pltpu.VMEM((NBUF, CHUNK, DSZ), jnp.float32),
            pltpu.SemaphoreType.DMA((NBUF,)),
            pltpu.SemaphoreType.DMA((NBUF,)),
        ],
    )(idx, w)


def kernel(x, W):
    idx = x.reshape(NW, NCHUNK, CHUNK).astype(jnp.int32)
    out = _emb_lookup(idx, W)
    return out.reshape(x.shape[0], x.shape[1], DSZ)
